# baseline (device time: 6764 ns/iter reference)
import jax
import jax.numpy as jnp
from jax import lax
from jax.experimental import pallas as pl
from jax.experimental.pallas import tpu as pltpu

C = 2


def kernel(x):
    m, n = x.shape
    rows = m // C

    def body(x_ref, out_ref, send_buf, comm_ref, sx, rx):
        my_x = lax.axis_index("x")
        my_y = lax.axis_index("y")
        partner = (1 - my_x, my_y)

        barrier_sem = pltpu.get_barrier_semaphore()
        pl.semaphore_signal(
            barrier_sem, inc=1,
            device_id=partner, device_id_type=pl.DeviceIdType.MESH,
        )
        for c in range(C):
            send_buf[c, :, :] = x_ref[pl.ds(c * rows, rows), :].astype(
                jnp.bfloat16
            )
        pl.semaphore_wait(barrier_sem, 1)

        rdmas = []
        for c in range(C):
            rdma = pltpu.make_async_remote_copy(
                src_ref=send_buf.at[c],
                dst_ref=comm_ref.at[c],
                send_sem=sx.at[c],
                recv_sem=rx.at[c],
                device_id=partner,
                device_id_type=pl.DeviceIdType.MESH,
            )
            rdma.start()
            rdmas.append(rdma)

        for c in range(C):
            rdmas[c].wait_recv()
            out_ref[pl.ds(c * rows, rows), :] = (
                x_ref[pl.ds(c * rows, rows), :]
                + comm_ref[c, :, :].astype(jnp.float32)
            )
        for c in range(C):
            rdmas[c].wait_send()

    return pl.pallas_call(
        body,
        out_shape=jax.ShapeDtypeStruct((m, n), x.dtype),
        in_specs=[pl.BlockSpec(memory_space=pltpu.VMEM)],
        out_specs=pl.BlockSpec(memory_space=pltpu.VMEM),
        scratch_shapes=[
            pltpu.VMEM((C, rows, n), jnp.bfloat16),
            pltpu.VMEM((C, rows, n), jnp.bfloat16),
            pltpu.SemaphoreType.DMA((C,)),
            pltpu.SemaphoreType.DMA((C,)),
        ],
        compiler_params=pltpu.CompilerParams(collective_id=0),
    )(x)
